# baseline (speedup 1.0000x reference)
import functools

import jax
import jax.numpy as jnp
from jax.experimental import pallas as pl

_BLOCK = 2048
_E = 8
_BIG_I = 127
_NT = (((1,), (1,)), ((), ()))


def _route(x, w, bt):
    s = jax.lax.dot_general(x, w, _NT, preferred_element_type=jnp.float32)
    st = s.T + bt
    iota = jax.lax.broadcasted_iota(jnp.int32, st.shape, 0)
    m1 = jnp.max(st, axis=0, keepdims=True)
    i1 = jnp.min(jnp.where(st == m1, iota, _BIG_I), axis=0, keepdims=True)
    masked = jnp.where(iota == i1, -jnp.inf, st)
    m2 = jnp.max(masked, axis=0, keepdims=True)
    i2 = jnp.min(jnp.where(masked == m2, iota, _BIG_I), axis=0, keepdims=True)
    rz = 1.0 / jnp.sum(jnp.exp(st - m1), axis=0, keepdims=True)
    return rz, jnp.exp(m2 - m1) * rz, i1, i2


def _gate_kernel(rgb_ref, ir_ref, w_rgb_ref, b_rgb_ref, w_ir_ref, b_ir_ref,
                 probs_ref, idx_rgb_ref, idx_ir_ref):
    bt_rgb = b_rgb_ref[...].T
    bt_ir = b_ir_ref[...].T
    p1r, p2r, i1r, i2r = _route(rgb_ref[...], w_rgb_ref[...], bt_rgb)
    p1i, p2i, i1i, i2i = _route(ir_ref[...], w_ir_ref[...], bt_ir)

    e1r, e2r = jnp.exp(p1r), jnp.exp(p2r)
    e1i, e2i = jnp.exp(p1i), jnp.exp(p2i)
    rden = 1.0 / (e1r + e2r + e1i + e2i)
    probs = jnp.concatenate([e1r, e2r, e1i, e2i], axis=0) * rden

    packed = jnp.concatenate(
        [jax.lax.bitcast_convert_type(probs, jnp.int32),
         i1r, i2r, i1i, i2i], axis=0)
    packed_t = packed.T
    probs_ref[...] = jax.lax.bitcast_convert_type(packed_t[:, 0:4], jnp.float32)
    idx_rgb_ref[...] = packed_t[:, 4:6]
    idx_ir_ref[...] = packed_t[:, 6:8]


@functools.partial(jax.jit, static_argnames=("interpret",))
def kernel(rgb_local, ir_local, W_rgb, b_rgb, W_ir, b_ir, interpret=False):
    n = rgb_local.shape[0]
    d = rgb_local.shape[1]
    grid = n // _BLOCK

    row_spec = pl.BlockSpec((_BLOCK, d), lambda i: (i, 0))
    w_spec = pl.BlockSpec((_E, d), lambda i: (0, 0))
    b_spec = pl.BlockSpec((1, _E), lambda i: (0, 0))

    return pl.pallas_call(
        _gate_kernel,
        grid=(grid,),
        in_specs=[row_spec, row_spec, w_spec, b_spec, w_spec, b_spec],
        out_specs=[
            pl.BlockSpec((_BLOCK, 4), lambda i: (i, 0)),
            pl.BlockSpec((_BLOCK, 2), lambda i: (i, 0)),
            pl.BlockSpec((_BLOCK, 2), lambda i: (i, 0)),
        ],
        out_shape=[
            jax.ShapeDtypeStruct((n, 4), jnp.float32),
            jax.ShapeDtypeStruct((n, 2), jnp.int32),
            jax.ShapeDtypeStruct((n, 2), jnp.int32),
        ],
        interpret=interpret,
    )(rgb_local, ir_local, W_rgb, b_rgb.reshape(1, _E),
      W_ir, b_ir.reshape(1, _E))


# speedup vs baseline: 1.6158x; 1.6158x over previous
import functools

import jax
import jax.numpy as jnp
from jax.experimental import pallas as pl

_BLOCK = 4096
_E = 8
_BIG_I = 127
_NT = (((1,), (1,)), ((), ()))


def _route(x, w, bt):
    s = jax.lax.dot_general(x, w, _NT, preferred_element_type=jnp.float32)
    st = s.T + bt
    iota = jax.lax.broadcasted_iota(jnp.int32, st.shape, 0)
    m1 = jnp.max(st, axis=0, keepdims=True)
    i1 = jnp.min(jnp.where(st == m1, iota, _BIG_I), axis=0, keepdims=True)
    masked = jnp.where(iota == i1, -jnp.inf, st)
    m2 = jnp.max(masked, axis=0, keepdims=True)
    i2 = jnp.min(jnp.where(masked == m2, iota, _BIG_I), axis=0, keepdims=True)
    rz = 1.0 / jnp.sum(jnp.exp(st - m1), axis=0, keepdims=True)
    return rz, jnp.exp(m2 - m1) * rz, i1, i2


def _gate_kernel(rgb_ref, ir_ref, w_rgb_ref, b_rgb_ref, w_ir_ref, b_ir_ref,
                 probs_ref, idx_rgb_ref, idx_ir_ref):
    p1r, p2r, i1r, i2r = _route(rgb_ref[...], w_rgb_ref[...], b_rgb_ref[...].T)
    p1i, p2i, i1i, i2i = _route(ir_ref[...], w_ir_ref[...], b_ir_ref[...].T)

    e1r, e2r = jnp.exp(p1r), jnp.exp(p2r)
    e1i, e2i = jnp.exp(p1i), jnp.exp(p2i)
    rden = 1.0 / (e1r + e2r + e1i + e2i)
    probs_ref[...] = jnp.concatenate([e1r, e2r, e1i, e2i], axis=0) * rden
    idx_rgb_ref[...] = jnp.concatenate([i1r, i2r], axis=0)
    idx_ir_ref[...] = jnp.concatenate([i1i, i2i], axis=0)


@functools.partial(jax.jit, static_argnames=("interpret",))
def kernel(rgb_local, ir_local, W_rgb, b_rgb, W_ir, b_ir, interpret=False):
    n = rgb_local.shape[0]
    d = rgb_local.shape[1]
    grid = n // _BLOCK

    row_spec = pl.BlockSpec((_BLOCK, d), lambda i: (i, 0))
    w_spec = pl.BlockSpec((_E, d), lambda i: (0, 0))
    b_spec = pl.BlockSpec((1, _E), lambda i: (0, 0))

    probs_t, idx_rgb_t, idx_ir_t = pl.pallas_call(
        _gate_kernel,
        grid=(grid,),
        in_specs=[row_spec, row_spec, w_spec, b_spec, w_spec, b_spec],
        out_specs=[
            pl.BlockSpec((4, _BLOCK), lambda i: (0, i)),
            pl.BlockSpec((2, _BLOCK), lambda i: (0, i)),
            pl.BlockSpec((2, _BLOCK), lambda i: (0, i)),
        ],
        out_shape=[
            jax.ShapeDtypeStruct((4, n), jnp.float32),
            jax.ShapeDtypeStruct((2, n), jnp.int32),
            jax.ShapeDtypeStruct((2, n), jnp.int32),
        ],
        interpret=interpret,
    )(rgb_local, ir_local, W_rgb, b_rgb.reshape(1, _E),
      W_ir, b_ir.reshape(1, _E))
    return probs_t.T, idx_rgb_t.T, idx_ir_t.T


# speedup vs baseline: 1.7143x; 1.0610x over previous
import functools

import jax
import jax.numpy as jnp
from jax.experimental import pallas as pl

_BLOCK = 2048
_E = 8
_BIG_I = 127
_NT = (((1,), (1,)), ((), ()))


def _route(x, w, bt):
    s = jax.lax.dot_general(x, w, _NT, preferred_element_type=jnp.float32)
    st = s.T + bt
    iota = jax.lax.broadcasted_iota(jnp.int32, st.shape, 0)
    m1 = jnp.max(st, axis=0, keepdims=True)
    i1 = jnp.min(jnp.where(st == m1, iota, _BIG_I), axis=0, keepdims=True)
    masked = jnp.where(iota == i1, -jnp.inf, st)
    m2 = jnp.max(masked, axis=0, keepdims=True)
    i2 = jnp.min(jnp.where(masked == m2, iota, _BIG_I), axis=0, keepdims=True)
    rz = 1.0 / jnp.sum(jnp.exp(st - m1), axis=0, keepdims=True)
    return rz, jnp.exp(m2 - m1) * rz, i1, i2


def _gate_kernel(rgb_ref, ir_ref, w_rgb_ref, b_rgb_ref, w_ir_ref, b_ir_ref,
                 probs_ref, idx_rgb_ref, idx_ir_ref):
    p1r, p2r, i1r, i2r = _route(rgb_ref[...], w_rgb_ref[...], b_rgb_ref[...].T)
    p1i, p2i, i1i, i2i = _route(ir_ref[...], w_ir_ref[...], b_ir_ref[...].T)

    e1r, e2r = jnp.exp(p1r), jnp.exp(p2r)
    e1i, e2i = jnp.exp(p1i), jnp.exp(p2i)
    rden = 1.0 / (e1r + e2r + e1i + e2i)
    probs_ref[...] = jnp.concatenate([e1r, e2r, e1i, e2i], axis=0) * rden
    idx_rgb_ref[...] = jnp.concatenate([i1r, i2r], axis=0)
    idx_ir_ref[...] = jnp.concatenate([i1i, i2i], axis=0)


@functools.partial(jax.jit, static_argnames=("interpret",))
def kernel(rgb_local, ir_local, W_rgb, b_rgb, W_ir, b_ir, interpret=False):
    n = rgb_local.shape[0]
    d = rgb_local.shape[1]
    grid = n // _BLOCK

    row_spec = pl.BlockSpec((_BLOCK, d), lambda i: (i, 0))
    w_spec = pl.BlockSpec((_E, d), lambda i: (0, 0))
    b_spec = pl.BlockSpec((1, _E), lambda i: (0, 0))

    probs_t, idx_rgb_t, idx_ir_t = pl.pallas_call(
        _gate_kernel,
        grid=(grid,),
        in_specs=[row_spec, row_spec, w_spec, b_spec, w_spec, b_spec],
        out_specs=[
            pl.BlockSpec((4, _BLOCK), lambda i: (0, i)),
            pl.BlockSpec((2, _BLOCK), lambda i: (0, i)),
            pl.BlockSpec((2, _BLOCK), lambda i: (0, i)),
        ],
        out_shape=[
            jax.ShapeDtypeStruct((4, n), jnp.float32),
            jax.ShapeDtypeStruct((2, n), jnp.int32),
            jax.ShapeDtypeStruct((2, n), jnp.int32),
        ],
        interpret=interpret,
    )(rgb_local, ir_local, W_rgb, b_rgb.reshape(1, _E),
      W_ir, b_ir.reshape(1, _E))
    return probs_t.T, idx_rgb_t.T, idx_ir_t.T
